# Initial kernel scaffold; baseline (speedup 1.0000x reference)
import jax
import jax.numpy as jnp
from jax.experimental import pallas as pl


def kernel(x, tables):
    raise NotImplementedError("write your pallas kernel here")


# speedup vs baseline: 4.5804x; 4.5804x over previous
import functools

import jax
import jax.numpy as jnp
from jax import lax
from jax.experimental import pallas as pl
from jax.experimental.pallas import tpu as pltpu
from jax.experimental.pallas import tpu_sc as plsc

NIN = 26
VOCAB_P1 = 1001
D_MODEL = 64

_info = plsc.get_sparse_core_info()
_NC, _NS, _LANES = _info.num_cores, _info.num_subcores, _info.num_lanes
_NW = _NC * _NS


def _make_emb(rows_total: int):
    rpw = rows_total // _NW
    chunk = 128
    nchunks = rpw // chunk
    mesh = plsc.VectorSubcoreMesh(core_axis_name="c", subcore_axis_name="s")

    @functools.partial(
        pl.kernel,
        mesh=mesh,
        out_type=jax.ShapeDtypeStruct((rows_total, D_MODEL), jnp.float32),
        scratch_types=[
            pltpu.VMEM((rpw,), jnp.int32),
            pltpu.VMEM((chunk, D_MODEL), jnp.float32),
            pltpu.SemaphoreType.DMA,
        ],
        compiler_params=pltpu.CompilerParams(use_tc_tiling_on_sc=False),
    )
    def emb(x_hbm, tab_hbm, out_hbm, gidx_v, rows_v, gsem):
        wid = lax.axis_index("s") * _NC + lax.axis_index("c")
        base = wid * rpw
        pltpu.sync_copy(x_hbm.at[pl.ds(base, rpw)], gidx_v)

        lane = lax.iota(jnp.int32, 16)

        def add_field_offset(j, carry):
            p = j * 16
            off = ((lane + p) % NIN) * VOCAB_P1
            gidx_v[pl.ds(p, 16)] = gidx_v[pl.ds(p, 16)] + off
            return carry

        lax.fori_loop(0, rpw // 16, add_field_offset, 0)

        def gather_chunk(c, carry):
            rb = c * chunk
            pltpu.async_copy(
                tab_hbm.at[gidx_v.at[pl.ds(rb, chunk)]], rows_v, gsem
            ).wait()
            pltpu.sync_copy(rows_v, out_hbm.at[pl.ds(base + rb, chunk)])
            return carry

        lax.fori_loop(0, nchunks, gather_chunk, 0)

    return emb


def kernel(x, tables):
    b, l, nin = x.shape
    rows_total = b * l * nin
    x_flat = x.reshape(rows_total).astype(jnp.int32)
    tab_flat = tables.reshape(nin * tables.shape[1], tables.shape[2])
    out = _make_emb(rows_total)(x_flat, tab_flat)
    return out.reshape(b, l, nin, D_MODEL)


# speedup vs baseline: 5.4279x; 1.1850x over previous
import functools

import jax
import jax.numpy as jnp
from jax import lax
from jax.experimental import pallas as pl
from jax.experimental.pallas import tpu as pltpu
from jax.experimental.pallas import tpu_sc as plsc

NIN = 26
VOCAB_P1 = 1001
D_MODEL = 64

_info = plsc.get_sparse_core_info()
_NC, _NS, _LANES = _info.num_cores, _info.num_subcores, _info.num_lanes
_NW = _NC * _NS


def _make_emb(rows_total: int):
    rpw = rows_total // _NW
    chunk = 128
    nchunks = rpw // chunk
    k = 5
    ngroups = nchunks // k
    npairs = ngroups // 2
    mesh = plsc.VectorSubcoreMesh(core_axis_name="c", subcore_axis_name="s")

    @functools.partial(
        pl.kernel,
        mesh=mesh,
        out_type=jax.ShapeDtypeStruct((rows_total, D_MODEL), jnp.float32),
        scratch_types=[
            pltpu.VMEM((rpw,), jnp.int32),
            pltpu.VMEM((k, chunk, D_MODEL), jnp.float32),
            pltpu.VMEM((k, chunk, D_MODEL), jnp.float32),
            pltpu.SemaphoreType.DMA,
            pltpu.SemaphoreType.DMA,
            pltpu.SemaphoreType.DMA,
            pltpu.SemaphoreType.DMA,
        ],
        compiler_params=pltpu.CompilerParams(use_tc_tiling_on_sc=False),
    )
    def emb(x_hbm, tab_hbm, out_hbm, gidx_v, buf_a, buf_b,
            gsem_a, gsem_b, osem_a, osem_b):
        wid = lax.axis_index("s") * _NC + lax.axis_index("c")
        base = wid * rpw
        pltpu.sync_copy(x_hbm.at[pl.ds(base, rpw)], gidx_v)

        lane = lax.iota(jnp.int32, 16)

        def add_field_offset(j, carry):
            p = j * 16
            off = ((lane + p) % NIN) * VOCAB_P1
            gidx_v[pl.ds(p, 16)] = gidx_v[pl.ds(p, 16)] + off
            return carry

        lax.fori_loop(0, rpw // 16, add_field_offset, 0)

        def g_copy(c, buf, j, sem):
            return pltpu.make_async_copy(
                tab_hbm.at[gidx_v.at[pl.ds(c * chunk, chunk)]], buf.at[j], sem)

        def s_copy(c, buf, j, sem):
            return pltpu.make_async_copy(
                buf.at[j], out_hbm.at[pl.ds(base + c * chunk, chunk)], sem)

        for j in range(k):
            g_copy(j, buf_a, j, gsem_a).start()

        def pair(p, carry):
            ga = 2 * p
            gb = ga + 1
            for j in range(k):
                g_copy(ga * k + j, buf_a, j, gsem_a).wait()
            for j in range(k):
                s_copy(ga * k + j, buf_a, j, osem_a).start()
            @pl.when(p > 0)
            def _():
                for j in range(k):
                    s_copy((gb - 2) * k + j, buf_b, j, osem_b).wait()
            for j in range(k):
                g_copy(gb * k + j, buf_b, j, gsem_b).start()
            for j in range(k):
                g_copy(gb * k + j, buf_b, j, gsem_b).wait()
            for j in range(k):
                s_copy(gb * k + j, buf_b, j, osem_b).start()
            for j in range(k):
                s_copy(ga * k + j, buf_a, j, osem_a).wait()
            @pl.when(p < npairs - 1)
            def _():
                for j in range(k):
                    g_copy((ga + 2) * k + j, buf_a, j, gsem_a).start()
            return carry

        lax.fori_loop(0, npairs, pair, 0)
        for j in range(k):
            s_copy((ngroups - 1) * k + j, buf_b, j, osem_b).wait()

    return emb


def kernel(x, tables):
    b, l, nin = x.shape
    rows_total = b * l * nin
    x_flat = x.reshape(rows_total).astype(jnp.int32)
    tab_flat = tables.reshape(nin * tables.shape[1], tables.shape[2])
    out = _make_emb(rows_total)(x_flat, tab_flat)
    return out.reshape(b, l, nin, D_MODEL)
